# baseline (speedup 1.0000x reference)
import functools

import jax
import jax.numpy as jnp
from jax import lax
from jax.experimental import pallas as pl
from jax.experimental.pallas import tpu as pltpu
from jax.experimental.pallas import tpu_sc as plsc

_MARGIN = 0.5
_EPS = 1e-8
_NBUF = 2
_PIECE = 256


def _normalize_body(e_ref, o_ref):
    e = e_ref[...]
    s = jnp.sum(e * e, axis=1, keepdims=True)
    n = jnp.maximum(jnp.sqrt(s), _EPS)
    eh = (e / n).astype(jnp.bfloat16)
    d2 = e.shape[1] // 2
    lo = jax.lax.bitcast_convert_type(
        eh[:, :d2], jnp.uint16).astype(jnp.uint32)
    hi = jax.lax.bitcast_convert_type(
        eh[:, d2:], jnp.uint16).astype(jnp.uint32)
    o_ref[...] = jax.lax.bitcast_convert_type(
        lo | (hi << 16), jnp.float32)


def _make_sc_loss(n_pairs, n_rows_pad, d_words, chunk, cpw, nc, ns):
    nw = nc * ns
    mesh = plsc.VectorSubcoreMesh(core_axis_name="c", subcore_axis_name="s")
    groups = chunk // 16
    dchunks = d_words // 16
    ppw = cpw * chunk
    tail_w = n_pairs // ppw
    tail_rows = n_pairs - tail_w * ppw
    rpt = (n_rows_pad // ns) // 8 * 8
    rpt_last = n_rows_pad - rpt * (ns - 1)

    row_bufs = [pltpu.VMEM((chunk, d_words), jnp.float32)
                for _ in range(2 * _NBUF)]
    sems = [pltpu.SemaphoreType.DMA for _ in range(2 * _NBUF)]

    @functools.partial(
        pl.kernel,
        mesh=mesh,
        compiler_params=pltpu.CompilerParams(
            use_tc_tiling_on_sc=False, needs_layout_passes=False),
        out_type=jax.ShapeDtypeStruct((nw, 8, 16), jnp.float32),
        scratch_types=[
            pltpu.VMEM_SHARED((n_rows_pad, d_words), jnp.float32),
            pltpu.VMEM((ppw,), jnp.int32),
            pltpu.VMEM((ppw,), jnp.int32),
            pltpu.VMEM((ppw,), jnp.int32),
            pltpu.VMEM((_PIECE, 3), jnp.int32),
            pltpu.VMEM((16, 16), jnp.float32),
            pltpu.VMEM((8, 16), jnp.float32),
        ] + row_bufs + sems,
    )
    def sc_loss(emb, pairs_r, out, table_s, idx1f, idx2f, labf, ptmp,
                dred, acc_v, *bufs_and_sems):
        r1 = bufs_and_sems[0:_NBUF]
        r2 = bufs_and_sems[_NBUF:2 * _NBUF]
        s1 = bufs_and_sems[2 * _NBUF:3 * _NBUF]
        s2 = bufs_and_sems[3 * _NBUF:4 * _NBUF]
        cid = lax.axis_index("c")
        sid = lax.axis_index("s")
        wid = sid * nc + cid
        lanes = lax.broadcasted_iota(jnp.int32, (16,), 0)

        pbase = wid * ppw

        zer16i = jnp.zeros((16,), jnp.int32)
        one16i = jnp.full((16,), 1, jnp.int32)
        two16i = jnp.full((16,), 2, jnp.int32)

        def stage(npieces):
            def piece_body(q, carry):
                pltpu.sync_copy(
                    pairs_r.at[pl.ds(pbase + q * _PIECE, _PIECE)], ptmp)

                def ex(i, c2):
                    rowv = i * 16 + lanes
                    obase = q * _PIECE + i * 16
                    idx1f[pl.ds(obase, 16)] = plsc.load_gather(
                        ptmp, [rowv, zer16i])
                    idx2f[pl.ds(obase, 16)] = plsc.load_gather(
                        ptmp, [rowv, one16i])
                    labf[pl.ds(obase, 16)] = plsc.load_gather(
                        ptmp, [rowv, two16i])
                    return c2

                lax.fori_loop(0, _PIECE // 16, ex, 0)
                return carry

            lax.fori_loop(0, npieces, piece_body, 0)

        @pl.when(wid < tail_w)
        def _():
            stage(ppw // _PIECE)

        if tail_rows:
            @pl.when(wid == tail_w)
            def _():
                stage(tail_rows // _PIECE)

                def fill(i, carry):
                    rows = tail_rows + i * 16 + lanes
                    plsc.store_scatter(idx1f, [rows], rows)
                    plsc.store_scatter(idx2f, [rows], rows)
                    plsc.store_scatter(labf, [rows], one16i)
                    return carry

                lax.fori_loop(0, (ppw - tail_rows) // 16, fill, 0)

        trow = sid * rpt

        @pl.when(sid < ns - 1)
        def _():
            pltpu.sync_copy(emb.at[pl.ds(trow, rpt)],
                            table_s.at[pl.ds(trow, rpt)])

        @pl.when(sid == ns - 1)
        def _():
            pltpu.sync_copy(emb.at[pl.ds(trow, rpt_last)],
                            table_s.at[pl.ds(trow, rpt_last)])

        plsc.subcore_barrier()

        zero16 = jnp.zeros((16,), jnp.float32)

        def issue(j, b):
            pltpu.async_copy(
                table_s.at[idx1f.at[pl.ds(j * chunk, chunk)]], r1[b], s1[b])
            pltpu.async_copy(
                table_s.at[idx2f.at[pl.ds(j * chunk, chunk)]], r2[b], s2[b])

        def wait(j, b):
            pltpu.make_async_copy(
                table_s.at[idx1f.at[pl.ds(j * chunk, chunk)]],
                r1[b], s1[b]).wait()
            pltpu.make_async_copy(
                table_s.at[idx2f.at[pl.ds(j * chunk, chunk)]],
                r2[b], s2[b]).wait()

        def compute(j, b, acc):
            ra, rb = r1[b], r2[b]

            def group_body(g, acc):
                for p16 in range(16):
                    p = g * 16 + p16
                    a = zero16
                    bb = zero16
                    for t in range(dchunks):
                        w1 = plsc.bitcast(ra[p, pl.ds(16 * t, 16)],
                                          jnp.bfloat16)
                        w2 = plsc.bitcast(rb[p, pl.ds(16 * t, 16)],
                                          jnp.bfloat16)
                        u1, v1 = plsc.unpack(
                            w1, format=plsc.PackFormat.INTERLEAVED)
                        u2, v2 = plsc.unpack(
                            w2, format=plsc.PackFormat.INTERLEAVED)
                        a = a + u1 * u2
                        bb = bb + v1 * v2
                    dred[p16, :] = a + bb
                dots = plsc.load_gather(
                    dred, [lanes, jnp.zeros((16,), jnp.int32)])
                for c in range(1, 16):
                    dots = dots + plsc.load_gather(
                        dred, [lanes, jnp.full((16,), c, jnp.int32)])
                l = labf[pl.ds(j * chunk + g * 16, 16)].astype(jnp.float32)
                t = 0.5 * (dots + 1.0)
                clamped = jnp.maximum(_MARGIN - t, 0.0)
                loss = (1.0 - l) * t * t + l * clamped * clamped
                return acc + loss

            return lax.fori_loop(0, groups, group_body, acc)

        for b in range(_NBUF):
            issue(b, b)

        def ring_body(jj, acc):
            for b in range(_NBUF):
                j = _NBUF * jj + b
                wait(j, b)
                acc = compute(j, b, acc)

                @pl.when(j + _NBUF < cpw)
                def _():
                    issue(j + _NBUF, b)

            return acc

        acc = lax.fori_loop(0, cpw // _NBUF, ring_body, zero16)
        acc_v[0, :] = acc
        for r in range(1, 8):
            acc_v[r, :] = zero16
        pltpu.sync_copy(acc_v, out.at[wid])

    return sc_loss


def kernel(embeddings, pairs):
    n_nodes, d_feat = embeddings.shape
    n_pairs = pairs.shape[0]
    info = plsc.get_sparse_core_info()
    nc, ns = info.num_cores, info.num_subcores
    nw = nc * ns
    chunk = 32
    per = nw * chunk
    cpw = -(-n_pairs // per)
    cpw = -(-cpw // 8) * 8
    np_pad = cpw * per

    rblk = n_nodes // 5
    d_words = d_feat // 2
    norm = pl.pallas_call(
        _normalize_body,
        out_shape=jax.ShapeDtypeStruct((n_nodes, d_words), jnp.float32),
        grid=(5,),
        in_specs=[pl.BlockSpec((rblk, d_feat), lambda i: (i, 0))],
        out_specs=pl.BlockSpec((rblk, d_words), lambda i: (i, 0)),
    )(embeddings)
    n_rows_pad = n_nodes

    sc_loss = _make_sc_loss(n_pairs, n_rows_pad, d_words, chunk, cpw, nc, ns)
    partials = sc_loss(norm, pairs)
    return jnp.sum(partials) / jnp.float32(n_pairs)


# speedup vs baseline: 1.8468x; 1.8468x over previous
import functools

import jax
import jax.numpy as jnp
from jax import lax
from jax.experimental import pallas as pl
from jax.experimental.pallas import tpu as pltpu
from jax.experimental.pallas import tpu_sc as plsc

_MARGIN = 0.5
_EPS = 1e-8
_NBUF = 2
_PIECE = 256


def _normalize_body(e_ref, o_ref):
    e = e_ref[...]
    s = jnp.sum(e * e, axis=1, keepdims=True)
    n = jnp.maximum(jnp.sqrt(s), _EPS)
    eh = (e / n).astype(jnp.bfloat16)
    d2 = e.shape[1] // 2
    lo = jax.lax.bitcast_convert_type(
        eh[:, :d2], jnp.uint16).astype(jnp.uint32)
    hi = jax.lax.bitcast_convert_type(
        eh[:, d2:], jnp.uint16).astype(jnp.uint32)
    o_ref[...] = jax.lax.bitcast_convert_type(
        lo | (hi << 16), jnp.float32)


def _make_sc_loss(n_pairs, n_rows_pad, d_words, chunk, cpw, nc, ns):
    nw = nc * ns
    mesh = plsc.VectorSubcoreMesh(core_axis_name="c", subcore_axis_name="s")
    groups = chunk // 16
    dchunks = d_words // 16
    ppw = cpw * chunk
    tail_w = n_pairs // ppw
    tail_rows = n_pairs - tail_w * ppw
    rpt = (n_rows_pad // ns) // 8 * 8
    rpt_last = n_rows_pad - rpt * (ns - 1)

    row_bufs = [pltpu.VMEM((chunk, d_words), jnp.float32)
                for _ in range(2 * _NBUF)]
    sems = [pltpu.SemaphoreType.DMA for _ in range(2 * _NBUF)]

    @functools.partial(
        pl.kernel,
        mesh=mesh,
        compiler_params=pltpu.CompilerParams(
            use_tc_tiling_on_sc=False, needs_layout_passes=False),
        out_type=jax.ShapeDtypeStruct((nw, 8, 16), jnp.float32),
        scratch_types=[
            pltpu.VMEM_SHARED((n_rows_pad, d_words), jnp.float32),
            pltpu.VMEM((cpw, chunk), jnp.int32),
            pltpu.VMEM((cpw, chunk), jnp.int32),
            pltpu.VMEM((cpw, chunk), jnp.int32),
            pltpu.VMEM((16, 16), jnp.float32),
            pltpu.VMEM((8, 16), jnp.float32),
        ] + row_bufs + sems,
    )
    def sc_loss(emb, idx1, idx2, labels, out, table_s, idx1_v, idx2_v, lab_v,
                dred, acc_v, *bufs_and_sems):
        r1 = bufs_and_sems[0:_NBUF]
        r2 = bufs_and_sems[_NBUF:2 * _NBUF]
        s1 = bufs_and_sems[2 * _NBUF:3 * _NBUF]
        s2 = bufs_and_sems[3 * _NBUF:4 * _NBUF]
        cid = lax.axis_index("c")
        sid = lax.axis_index("s")
        wid = sid * nc + cid
        lanes = lax.broadcasted_iota(jnp.int32, (16,), 0)
        base = wid * cpw
        pltpu.sync_copy(idx1.at[pl.ds(base, cpw)], idx1_v)
        pltpu.sync_copy(idx2.at[pl.ds(base, cpw)], idx2_v)
        pltpu.sync_copy(labels.at[pl.ds(base, cpw)], lab_v)

        trow = sid * rpt

        @pl.when(sid < ns - 1)
        def _():
            pltpu.sync_copy(emb.at[pl.ds(trow, rpt)],
                            table_s.at[pl.ds(trow, rpt)])

        @pl.when(sid == ns - 1)
        def _():
            pltpu.sync_copy(emb.at[pl.ds(trow, rpt_last)],
                            table_s.at[pl.ds(trow, rpt_last)])

        plsc.subcore_barrier()

        zero16 = jnp.zeros((16,), jnp.float32)

        def issue(j, b):
            pltpu.async_copy(table_s.at[idx1_v.at[j]], r1[b], s1[b])
            pltpu.async_copy(table_s.at[idx2_v.at[j]], r2[b], s2[b])

        def wait(j, b):
            pltpu.make_async_copy(
                table_s.at[idx1_v.at[j]], r1[b], s1[b]).wait()
            pltpu.make_async_copy(
                table_s.at[idx2_v.at[j]], r2[b], s2[b]).wait()

        def compute(j, b, acc):
            ra, rb = r1[b], r2[b]

            def group_body(g, acc):
                for p16 in range(16):
                    p = g * 16 + p16
                    a = zero16
                    bb = zero16
                    for t in range(dchunks):
                        w1 = plsc.bitcast(ra[p, pl.ds(16 * t, 16)],
                                          jnp.bfloat16)
                        w2 = plsc.bitcast(rb[p, pl.ds(16 * t, 16)],
                                          jnp.bfloat16)
                        u1, v1 = plsc.unpack(
                            w1, format=plsc.PackFormat.INTERLEAVED)
                        u2, v2 = plsc.unpack(
                            w2, format=plsc.PackFormat.INTERLEAVED)
                        a = a + u1 * u2
                        bb = bb + v1 * v2
                    dred[p16, :] = a + bb
                dots = plsc.load_gather(
                    dred, [lanes, jnp.zeros((16,), jnp.int32)])
                for c in range(1, 16):
                    dots = dots + plsc.load_gather(
                        dred, [lanes, jnp.full((16,), c, jnp.int32)])
                l = lab_v[j, pl.ds(g * 16, 16)].astype(jnp.float32)
                t = 0.5 * (dots + 1.0)
                clamped = jnp.maximum(_MARGIN - t, 0.0)
                loss = (1.0 - l) * t * t + l * clamped * clamped
                return acc + loss

            return lax.fori_loop(0, groups, group_body, acc)

        for b in range(_NBUF):
            issue(b, b)

        def ring_body(jj, acc):
            for b in range(_NBUF):
                j = _NBUF * jj + b
                wait(j, b)
                acc = compute(j, b, acc)

                @pl.when(j + _NBUF < cpw)
                def _():
                    issue(j + _NBUF, b)

            return acc

        acc = lax.fori_loop(0, cpw // _NBUF, ring_body, zero16)
        acc_v[0, :] = acc
        for r in range(1, 8):
            acc_v[r, :] = zero16
        pltpu.sync_copy(acc_v, out.at[wid])

    return sc_loss


def kernel(embeddings, pairs):
    n_nodes, d_feat = embeddings.shape
    n_pairs = pairs.shape[0]
    info = plsc.get_sparse_core_info()
    nc, ns = info.num_cores, info.num_subcores
    nw = nc * ns
    chunk = 64
    per = nw * chunk
    cpw = -(-n_pairs // per)
    cpw = -(-cpw // 8) * 8
    np_pad = cpw * per

    rblk = n_nodes // 5
    d_words = d_feat // 2
    norm = pl.pallas_call(
        _normalize_body,
        out_shape=jax.ShapeDtypeStruct((n_nodes, d_words), jnp.float32),
        grid=(5,),
        in_specs=[pl.BlockSpec((rblk, d_feat), lambda i: (i, 0))],
        out_specs=pl.BlockSpec((rblk, d_words), lambda i: (i, 0)),
    )(embeddings)
    n_rows_pad = n_nodes

    pad = np_pad - n_pairs
    pad_idx = jnp.arange(pad, dtype=jnp.int32) % jnp.int32(n_nodes)
    idx1 = jnp.concatenate(
        [pairs[:, 0], pad_idx]).reshape(nw * cpw, chunk)
    idx2 = jnp.concatenate(
        [pairs[:, 1], pad_idx]).reshape(nw * cpw, chunk)
    lab = jnp.concatenate(
        [pairs[:, 2], jnp.ones((pad,), jnp.int32)]).reshape(nw * cpw, chunk)

    sc_loss = _make_sc_loss(n_pairs, n_rows_pad, d_words, chunk, cpw, nc, ns)
    partials = sc_loss(norm, idx1, idx2, lab)
    return jnp.sum(partials) / jnp.float32(n_pairs)
